# Initial kernel scaffold; baseline (speedup 1.0000x reference)
import functools

import jax
import jax.numpy as jnp
from jax import lax
from jax.experimental import pallas as pl
from jax.experimental.pallas import tpu as pltpu
from jax.experimental.pallas import tpu_sc as plsc

L = 16
NW = 32
ROW = 128
CHUNK_ROWS = 8


def _make_kernel(n_obs, n_points, n_cams):
    assert n_obs % ROW == 0
    n_rows = n_obs // ROW
    rows_base = n_rows // NW
    rows_extra = n_rows % NW
    chunk_obs = CHUNK_ROWS * ROW
    groups_per_chunk = chunk_obs // L

    mesh = plsc.VectorSubcoreMesh(core_axis_name="c", subcore_axis_name="s")

    @functools.partial(
        pl.kernel,
        mesh=mesh,
        out_type=jax.ShapeDtypeStruct((n_obs, 2), jnp.float32),
        scratch_types=[
            pltpu.VMEM((n_cams, 10), jnp.float32),
            pltpu.VMEM((CHUNK_ROWS, ROW), jnp.int32),
            pltpu.VMEM((chunk_obs,), jnp.int32),
            pltpu.VMEM((chunk_obs, 2), jnp.float32),
            pltpu.VMEM((chunk_obs, 3), jnp.float32),
            pltpu.VMEM((chunk_obs, 2), jnp.float32),
            pltpu.SemaphoreType.DMA,
        ],
    )
    def residual_kernel(obs_hbm, cidx_hbm, pidx_hbm, pts_hbm, cam_hbm,
                        out_hbm, cam_v, pidx_v, cidx_v, obs_v, pts_v,
                        out_v, sem):
        w = lax.axis_index("s") * 2 + lax.axis_index("c")
        my_rows = rows_base + jnp.where(w < rows_extra, 1, 0)
        row_base = rows_base * w + jnp.minimum(w, rows_extra)
        nchunks = (my_rows + CHUNK_ROWS - 1) // CHUNK_ROWS

        pltpu.sync_copy(cam_hbm, cam_v)

        col = [jnp.full((L,), k, jnp.int32) for k in range(10)]
        two = jnp.float32(2.0)

        def do_chunk(c, carry):
            rb = row_base + jnp.minimum(c * CHUNK_ROWS, my_rows - CHUNK_ROWS)
            ob = rb * ROW

            pltpu.sync_copy(pidx_hbm.at[pl.ds(rb, CHUNK_ROWS)], pidx_v)
            pltpu.sync_copy(cidx_hbm.at[pl.ds(ob, chunk_obs)], cidx_v)
            pltpu.sync_copy(obs_hbm.at[pl.ds(ob, chunk_obs)], obs_v)

            handles = []
            for j in range(CHUNK_ROWS):
                handles.append(pltpu.async_copy(
                    pts_hbm.at[pidx_v.at[j]],
                    pts_v.at[pl.ds(j * ROW, ROW)], sem))
            for h in handles:
                h.wait()

            def do_group(g, carry2):
                rows = g * L + lax.iota(jnp.int32, L)
                ci = cidx_v[pl.ds(g * L, L)]

                px = plsc.load_gather(pts_v, [rows, col[0]])
                py = plsc.load_gather(pts_v, [rows, col[1]])
                pz = plsc.load_gather(pts_v, [rows, col[2]])

                t0 = plsc.load_gather(cam_v, [ci, col[0]])
                t1 = plsc.load_gather(cam_v, [ci, col[1]])
                t2 = plsc.load_gather(cam_v, [ci, col[2]])
                qx = plsc.load_gather(cam_v, [ci, col[3]])
                qy = plsc.load_gather(cam_v, [ci, col[4]])
                qz = plsc.load_gather(cam_v, [ci, col[5]])
                qw = plsc.load_gather(cam_v, [ci, col[6]])
                fo = plsc.load_gather(cam_v, [ci, col[7]])
                k1 = plsc.load_gather(cam_v, [ci, col[8]])
                k2 = plsc.load_gather(cam_v, [ci, col[9]])

                ox = plsc.load_gather(obs_v, [rows, col[0]])
                oy = plsc.load_gather(obs_v, [rows, col[1]])

                uvx = qy * pz - qz * py
                uvy = qz * px - qx * pz
                uvz = qx * py - qy * px
                uuvx = qy * uvz - qz * uvy
                uuvy = qz * uvx - qx * uvz
                uuvz = qx * uvy - qy * uvx
                cpx = px + two * (qw * uvx + uuvx) + t0
                cpy = py + two * (qw * uvy + uuvy) + t1
                cpz = pz + two * (qw * uvz + uuvz) + t2

                inv = jnp.float32(-1.0) / cpz
                nx = cpx * inv
                ny = cpy * inv
                r2 = nx * nx + ny * ny
                dist = jnp.float32(1.0) + r2 * (k1 + r2 * k2)
                fd = fo * dist

                plsc.store_scatter(out_v, [rows, col[0]], fd * nx - ox)
                plsc.store_scatter(out_v, [rows, col[1]], fd * ny - oy)
                return carry2

            lax.fori_loop(0, groups_per_chunk, do_group, 0)
            pltpu.sync_copy(out_v, out_hbm.at[pl.ds(ob, chunk_obs)])
            return carry

        lax.fori_loop(0, nchunks, do_chunk, 0)

    return residual_kernel


def kernel(observes, cidx, pidx, points, camera_params):
    n_obs = observes.shape[0]
    n_points, _ = points.shape
    n_cams, _ = camera_params.shape
    pidx2 = pidx.reshape(n_obs // ROW, ROW).astype(jnp.int32)
    cidx = cidx.astype(jnp.int32)
    fn = _make_kernel(n_obs, n_points, n_cams)
    return fn(observes, cidx, pidx2, points, camera_params)


# speedup vs baseline: 3.9472x; 3.9472x over previous
import functools

import jax
import jax.numpy as jnp
from jax import lax
from jax.experimental import pallas as pl
from jax.experimental.pallas import tpu as pltpu
from jax.experimental.pallas import tpu_sc as plsc

L = 16
NW = 32
ROW = 128
CHUNK_ROWS = 8


def _make_kernel(n_obs, n_points, n_cams):
    assert n_obs % ROW == 0
    n_rows = n_obs // ROW
    rows_base = n_rows // NW
    rows_extra = n_rows % NW
    chunk_obs = CHUNK_ROWS * ROW
    groups_per_chunk = chunk_obs // L

    mesh = plsc.VectorSubcoreMesh(core_axis_name="c", subcore_axis_name="s")

    @functools.partial(
        pl.kernel,
        mesh=mesh,
        compiler_params=pltpu.CompilerParams(needs_layout_passes=False),
        out_type=jax.ShapeDtypeStruct((n_obs * 2,), jnp.float32),
        scratch_types=[
            pltpu.VMEM((n_cams * 10,), jnp.float32),
            pltpu.VMEM((chunk_obs,), jnp.int32),
            pltpu.VMEM((chunk_obs,), jnp.int32),
            pltpu.VMEM((chunk_obs * 2,), jnp.float32),
            pltpu.VMEM((chunk_obs,), jnp.float32),
            pltpu.VMEM((chunk_obs,), jnp.float32),
            pltpu.VMEM((chunk_obs,), jnp.float32),
            pltpu.VMEM((chunk_obs * 2,), jnp.float32),
            pltpu.SemaphoreType.DMA,
        ],
    )
    def residual_kernel(obs_hbm, cidx_hbm, pidx_hbm, px_hbm, py_hbm,
                        pz_hbm, cam_hbm, out_hbm, cam_v, pidx_v, cidx_v,
                        obs_v, px_v, py_v, pz_v, out_v, sem):
        w = lax.axis_index("s") * 2 + lax.axis_index("c")
        my_rows = rows_base + jnp.where(w < rows_extra, 1, 0)
        row_base = rows_base * w + jnp.minimum(w, rows_extra)
        nchunks = (my_rows + CHUNK_ROWS - 1) // CHUNK_ROWS

        pltpu.sync_copy(cam_hbm, cam_v)

        iota = lax.iota(jnp.int32, L)
        two = jnp.float32(2.0)

        def do_chunk(c, carry):
            rb = row_base + jnp.minimum(c * CHUNK_ROWS, my_rows - CHUNK_ROWS)
            ob = rb * ROW

            pltpu.sync_copy(pidx_hbm.at[pl.ds(ob, chunk_obs)], pidx_v)
            pltpu.sync_copy(cidx_hbm.at[pl.ds(ob, chunk_obs)], cidx_v)
            pltpu.sync_copy(obs_hbm.at[pl.ds(ob * 2, chunk_obs * 2)], obs_v)

            handles = []
            for j in range(CHUNK_ROWS):
                sl = pl.ds(j * ROW, ROW)
                idx = pidx_v.at[sl]
                handles.append(
                    pltpu.async_copy(px_hbm.at[idx], px_v.at[sl], sem))
                handles.append(
                    pltpu.async_copy(py_hbm.at[idx], py_v.at[sl], sem))
                handles.append(
                    pltpu.async_copy(pz_hbm.at[idx], pz_v.at[sl], sem))
            for h in handles:
                h.wait()

            def do_group(g, carry2):
                sl = pl.ds(g * L, L)
                rows2 = (g * L + iota) * 2
                ci10 = cidx_v[sl] * 10

                px = px_v[sl]
                py = py_v[sl]
                pz = pz_v[sl]

                t0 = plsc.load_gather(cam_v, [ci10])
                t1 = plsc.load_gather(cam_v, [ci10 + 1])
                t2 = plsc.load_gather(cam_v, [ci10 + 2])
                qx = plsc.load_gather(cam_v, [ci10 + 3])
                qy = plsc.load_gather(cam_v, [ci10 + 4])
                qz = plsc.load_gather(cam_v, [ci10 + 5])
                qw = plsc.load_gather(cam_v, [ci10 + 6])
                fo = plsc.load_gather(cam_v, [ci10 + 7])
                k1 = plsc.load_gather(cam_v, [ci10 + 8])
                k2 = plsc.load_gather(cam_v, [ci10 + 9])

                ox = plsc.load_gather(obs_v, [rows2])
                oy = plsc.load_gather(obs_v, [rows2 + 1])

                uvx = qy * pz - qz * py
                uvy = qz * px - qx * pz
                uvz = qx * py - qy * px
                uuvx = qy * uvz - qz * uvy
                uuvy = qz * uvx - qx * uvz
                uuvz = qx * uvy - qy * uvx
                cpx = px + two * (qw * uvx + uuvx) + t0
                cpy = py + two * (qw * uvy + uuvy) + t1
                cpz = pz + two * (qw * uvz + uuvz) + t2

                inv = jnp.float32(-1.0) / cpz
                nx = cpx * inv
                ny = cpy * inv
                r2 = nx * nx + ny * ny
                dist = jnp.float32(1.0) + r2 * (k1 + r2 * k2)
                fd = fo * dist

                plsc.store_scatter(out_v, [rows2], fd * nx - ox)
                plsc.store_scatter(out_v, [rows2 + 1], fd * ny - oy)
                return carry2

            lax.fori_loop(0, groups_per_chunk, do_group, 0)
            pltpu.sync_copy(out_v, out_hbm.at[pl.ds(ob * 2, chunk_obs * 2)])
            return carry

        lax.fori_loop(0, nchunks, do_chunk, 0)

    return residual_kernel


def kernel(observes, cidx, pidx, points, camera_params):
    n_obs = observes.shape[0]
    n_points, _ = points.shape
    n_cams, _ = camera_params.shape
    pts_t = points.T
    fn = _make_kernel(n_obs, n_points, n_cams)
    out = fn(observes.reshape(-1), cidx.astype(jnp.int32),
             pidx.astype(jnp.int32), pts_t[0], pts_t[1], pts_t[2],
             camera_params.reshape(-1))
    return out.reshape(n_obs, 2)
